# Initial kernel scaffold; baseline (speedup 1.0000x reference)
import jax
import jax.numpy as jnp
from jax.experimental import pallas as pl


def kernel(x, edge_index, W1, W2, b, Ws, bs):
    raise NotImplementedError("write your pallas kernel here")


# speedup vs baseline: 4.1800x; 4.1800x over previous
import functools

import jax
import jax.numpy as jnp
from jax import lax
from jax.experimental import pallas as pl
from jax.experimental.pallas import tpu as pltpu
from jax.experimental.pallas import tpu_sc as plsc

N = 10000
E = 160000
D = 256
C = 256
K = 16

NSC = 2
NSUB = 16
CH = C // NSC
EPS = E // NSUB
CHUNK = 80
NCHUNK = EPS // CHUNK
RPS = N // NSUB
ZROWS = 25


def _h_body(x_ref, w1_ref, h_ref):
    h_ref[...] = jnp.dot(x_ref[...], w1_ref[...],
                         preferred_element_type=jnp.float32)


def _h_split(x, W1, rb=400):
    nb = N // rb
    return pl.pallas_call(
        _h_body,
        grid=(nb, NSC),
        in_specs=[
            pl.BlockSpec((rb, D), lambda i, c: (i, 0)),
            pl.BlockSpec((D, CH), lambda i, c: (0, c)),
        ],
        out_specs=pl.BlockSpec((rb, CH), lambda i, c: (c * nb + i, 0)),
        out_shape=jax.ShapeDtypeStruct((NSC * N, CH), jnp.float32),
    )(x, W1)


def _sc_body(h_hbm, src_hbm, dst_hbm, out_hbm,
             agg_sh, src_v, dst_v, zbuf, rows_v):
    c = lax.axis_index("c")
    s = lax.axis_index("s")
    w = c * NSUB + s

    pltpu.sync_copy(src_hbm.at[w], src_v)
    pltpu.sync_copy(dst_hbm.at[s], dst_v)

    @pl.loop(0, ZROWS)
    def _(i):
        for k in range(CH // 16):
            zbuf[i, pl.ds(k * 16, 16)] = jnp.zeros((16,), jnp.float32)

    @pl.loop(0, RPS // ZROWS)
    def _(r):
        pltpu.sync_copy(zbuf, agg_sh.at[pl.ds(s * RPS + r * ZROWS, ZROWS)])

    plsc.subcore_barrier()

    @pl.loop(0, NCHUNK)
    def _(j):
        pltpu.sync_copy(h_hbm.at[src_v.at[j]], rows_v)
        pltpu.sync_copy(rows_v, agg_sh.at[dst_v.at[j]], add=True)

    plsc.subcore_barrier()

    wbase = s * 624
    pltpu.sync_copy(agg_sh.at[pl.ds(wbase, 624)],
                    out_hbm.at[pl.ds(c * N + wbase, 624)])

    @pl.when(s == NSUB - 1)
    def _():
        pltpu.sync_copy(agg_sh.at[pl.ds(624 * NSUB, N - 624 * NSUB)],
                        out_hbm.at[pl.ds(c * N + 624 * NSUB, N - 624 * NSUB)])


@functools.partial(
    pl.kernel,
    out_type=jax.ShapeDtypeStruct((NSC * N, CH), jnp.float32),
    mesh=plsc.VectorSubcoreMesh(core_axis_name="c", subcore_axis_name="s"),
    scratch_types=[
        pltpu.VMEM_SHARED((N, CH), jnp.float32),
        pltpu.VMEM((NCHUNK, CHUNK), jnp.int32),
        pltpu.VMEM((NCHUNK, CHUNK), jnp.int32),
        pltpu.VMEM((ZROWS, CH), jnp.float32),
        pltpu.VMEM((CHUNK, CH), jnp.float32),
    ],
)
def _sc_scatter(h_hbm, src_hbm, dst_hbm, out_hbm,
                agg_sh, src_v, dst_v, zbuf, rows_v):
    _sc_body(h_hbm, src_hbm, dst_hbm, out_hbm,
             agg_sh, src_v, dst_v, zbuf, rows_v)


def _out_body(agg_ref, x_ref, w2_ref, b_ref, ws_ref, bs_ref, o_ref):
    c = pl.program_id(1)
    t = agg_ref[...] + jnp.dot(x_ref[...], w2_ref[...],
                               preferred_element_type=jnp.float32) + b_ref[0]
    t = jnp.where(t > 0, t, jnp.exp(jnp.minimum(t, 0.0)) - 1.0)
    part = jnp.dot(t, ws_ref[0], preferred_element_type=jnp.float32)

    @pl.when(c == 0)
    def _():
        o_ref[...] = part

    @pl.when(c == 1)
    def _():
        z = o_ref[...] + part + bs_ref[...]
        m = jnp.max(z, axis=-1, keepdims=True)
        e = jnp.exp(z - m)
        o_ref[...] = e / jnp.sum(e, axis=-1, keepdims=True)


def _finish(agg_split, x, W2, b, Ws, bs, rb=400):
    nb = N // rb
    return pl.pallas_call(
        _out_body,
        grid=(nb, NSC),
        in_specs=[
            pl.BlockSpec((rb, CH), lambda i, c: (c * nb + i, 0)),
            pl.BlockSpec((rb, D), lambda i, c: (i, 0)),
            pl.BlockSpec((D, CH), lambda i, c: (0, c)),
            pl.BlockSpec((1, 1, CH), lambda i, c: (c, 0, 0)),
            pl.BlockSpec((1, CH, K), lambda i, c: (c, 0, 0)),
            pl.BlockSpec((1, K), lambda i, c: (0, 0)),
        ],
        out_specs=pl.BlockSpec((rb, K), lambda i, c: (i, 0)),
        out_shape=jax.ShapeDtypeStruct((N, K), jnp.float32),
    )(agg_split, x, W2, b.reshape(NSC, 1, CH), Ws.reshape(NSC, CH, K),
      bs.reshape(1, K))


def kernel(x, edge_index, W1, W2, b, Ws, bs):
    src = edge_index[0]
    dst = edge_index[1]
    srcr = src.reshape(NSUB, NCHUNK, CHUNK)
    src2 = jnp.concatenate([srcr, srcr + N]).reshape(NSC * NSUB, NCHUNK, CHUNK)
    dst2 = dst.reshape(NSUB, NCHUNK, CHUNK)

    h_split = _h_split(x, W1)
    agg_split = _sc_scatter(h_split, src2, dst2)
    return _finish(agg_split, x, W2, b, Ws, bs)


# speedup vs baseline: 4.4357x; 1.0612x over previous
import functools

import jax
import jax.numpy as jnp
from jax import lax
from jax.experimental import pallas as pl
from jax.experimental.pallas import tpu as pltpu
from jax.experimental.pallas import tpu_sc as plsc

N = 10000
E = 160000
D = 256
C = 256
K = 16

NSC = 2
NSUB = 16
CH = C // NSC
EPS = E // NSUB
CHUNK = 128
NCHUNK = -(-EPS // CHUNK)
EPP = NCHUNK * CHUNK
TRASH = N
AGG_ROWS = 10240
RPS = AGG_ROWS // NSUB


def _h_body(x_ref, w1_ref, h_ref):
    h_ref[...] = jnp.dot(x_ref[...], w1_ref[...],
                         preferred_element_type=jnp.float32)


def _h_split(x, W1, rb=400):
    nb = N // rb
    return pl.pallas_call(
        _h_body,
        grid=(nb, NSC),
        in_specs=[
            pl.BlockSpec((rb, D), lambda i, c: (i, 0)),
            pl.BlockSpec((D, CH), lambda i, c: (0, c)),
        ],
        out_specs=pl.BlockSpec((rb, CH), lambda i, c: (c * nb + i, 0)),
        out_shape=jax.ShapeDtypeStruct((NSC * N, CH), jnp.float32),
    )(x, W1)


def _sc_body(h_hbm, src_hbm, dst_hbm, out_hbm, agg_sh,
             sa, sb, da, db, rows0, rows1, sem_a, sem_b, sem0, sem1):
    c = lax.axis_index("c")
    s = lax.axis_index("s")
    sbase = (c * NSUB + s) * EPP
    dbase = s * EPP

    def idx_load(j, sref, dref, sem):
        pltpu.async_copy(src_hbm.at[pl.ds(sbase + j * CHUNK, CHUNK)], sref, sem)
        pltpu.async_copy(dst_hbm.at[pl.ds(dbase + j * CHUNK, CHUNK)], dref, sem)

    def idx_wait(j, sref, dref, sem):
        pltpu.make_async_copy(
            src_hbm.at[pl.ds(sbase + j * CHUNK, CHUNK)], sref, sem).wait()
        pltpu.make_async_copy(
            dst_hbm.at[pl.ds(dbase + j * CHUNK, CHUNK)], dref, sem).wait()

    def gather_wait(sref, rows, sem):
        pltpu.make_async_copy(h_hbm.at[sref], rows, sem).wait()

    @pl.loop(0, CHUNK)
    def _(i):
        for k in range(CH // 16):
            rows0[i, pl.ds(k * 16, 16)] = jnp.zeros((16,), jnp.float32)

    @pl.loop(0, RPS // CHUNK)
    def _(r):
        pltpu.sync_copy(rows0, agg_sh.at[pl.ds(s * RPS + r * CHUNK, CHUNK)])

    plsc.subcore_barrier()

    idx_load(0, sa, da, sem_a)
    idx_load(1, sb, db, sem_b)
    idx_wait(0, sa, da, sem_a)
    pltpu.async_copy(h_hbm.at[sa], rows0, sem0)

    @pl.loop(0, (NCHUNK - 1) // 2)
    def _(p):
        j = 2 * p
        idx_wait(j + 1, sb, db, sem_b)
        gather_wait(sa, rows0, sem0)
        pltpu.async_copy(h_hbm.at[sb], rows1, sem1)
        pltpu.sync_copy(rows0, agg_sh.at[da], add=True)
        idx_load(j + 2, sa, da, sem_a)
        idx_wait(j + 2, sa, da, sem_a)
        gather_wait(sb, rows1, sem1)
        pltpu.async_copy(h_hbm.at[sa], rows0, sem0)
        pltpu.sync_copy(rows1, agg_sh.at[db], add=True)

        @pl.when(j + 3 < NCHUNK)
        def _():
            idx_load(j + 3, sb, db, sem_b)

    gather_wait(sa, rows0, sem0)
    pltpu.sync_copy(rows0, agg_sh.at[da], add=True)

    plsc.subcore_barrier()

    wbase = s * 624
    pltpu.sync_copy(agg_sh.at[pl.ds(wbase, 624)],
                    out_hbm.at[pl.ds(c * N + wbase, 624)])

    @pl.when(s == NSUB - 1)
    def _():
        pltpu.sync_copy(agg_sh.at[pl.ds(624 * NSUB, N - 624 * NSUB)],
                        out_hbm.at[pl.ds(c * N + 624 * NSUB, N - 624 * NSUB)])


@functools.partial(
    pl.kernel,
    out_type=jax.ShapeDtypeStruct((NSC * N, CH), jnp.float32),
    mesh=plsc.VectorSubcoreMesh(core_axis_name="c", subcore_axis_name="s"),
    scratch_types=[
        pltpu.VMEM_SHARED((AGG_ROWS, CH), jnp.float32),
        pltpu.VMEM((CHUNK,), jnp.int32),
        pltpu.VMEM((CHUNK,), jnp.int32),
        pltpu.VMEM((CHUNK,), jnp.int32),
        pltpu.VMEM((CHUNK,), jnp.int32),
        pltpu.VMEM((CHUNK, CH), jnp.float32),
        pltpu.VMEM((CHUNK, CH), jnp.float32),
        pltpu.SemaphoreType.DMA,
        pltpu.SemaphoreType.DMA,
        pltpu.SemaphoreType.DMA,
        pltpu.SemaphoreType.DMA,
    ],
)
def _sc_scatter(h_hbm, src_hbm, dst_hbm, out_hbm, agg_sh,
                sa, sb, da, db, rows0, rows1, sem_a, sem_b, sem0, sem1):
    _sc_body(h_hbm, src_hbm, dst_hbm, out_hbm, agg_sh,
             sa, sb, da, db, rows0, rows1, sem_a, sem_b, sem0, sem1)


def _out_body(agg_ref, x_ref, w2_ref, b_ref, ws_ref, bs_ref, o_ref):
    c = pl.program_id(1)
    t = agg_ref[...] + jnp.dot(x_ref[...], w2_ref[...],
                               preferred_element_type=jnp.float32) + b_ref[0]
    t = jnp.where(t > 0, t, jnp.exp(jnp.minimum(t, 0.0)) - 1.0)
    part = jnp.dot(t, ws_ref[0], preferred_element_type=jnp.float32)

    @pl.when(c == 0)
    def _():
        o_ref[...] = part

    @pl.when(c == 1)
    def _():
        z = o_ref[...] + part + bs_ref[...]
        m = jnp.max(z, axis=-1, keepdims=True)
        e = jnp.exp(z - m)
        o_ref[...] = e / jnp.sum(e, axis=-1, keepdims=True)


def _finish(agg_split, x, W2, b, Ws, bs, rb=400):
    nb = N // rb
    return pl.pallas_call(
        _out_body,
        grid=(nb, NSC),
        in_specs=[
            pl.BlockSpec((rb, CH), lambda i, c: (c * nb + i, 0)),
            pl.BlockSpec((rb, D), lambda i, c: (i, 0)),
            pl.BlockSpec((D, CH), lambda i, c: (0, c)),
            pl.BlockSpec((1, 1, CH), lambda i, c: (c, 0, 0)),
            pl.BlockSpec((1, CH, K), lambda i, c: (c, 0, 0)),
            pl.BlockSpec((1, K), lambda i, c: (0, 0)),
        ],
        out_specs=pl.BlockSpec((rb, K), lambda i, c: (i, 0)),
        out_shape=jax.ShapeDtypeStruct((N, K), jnp.float32),
    )(agg_split, x, W2, b.reshape(NSC, 1, CH), Ws.reshape(NSC, CH, K),
      bs.reshape(1, K))


def kernel(x, edge_index, W1, W2, b, Ws, bs):
    src = edge_index[0]
    dst = edge_index[1]
    srcp = jnp.pad(src.reshape(NSUB, EPS), ((0, 0), (0, EPP - EPS)))
    dstp = jnp.pad(dst.reshape(NSUB, EPS), ((0, 0), (0, EPP - EPS)),
                   constant_values=TRASH)
    srcf = jnp.concatenate([srcp, srcp + N]).reshape(-1)
    dstf = dstp.reshape(-1)

    h_split = _h_split(x, W1)
    agg_split = _sc_scatter(h_split, srcf, dstf)
    return _finish(agg_split, x, W2, b, Ws, bs)


# speedup vs baseline: 4.7466x; 1.0701x over previous
import functools

import jax
import jax.numpy as jnp
from jax import lax
from jax.experimental import pallas as pl
from jax.experimental.pallas import tpu as pltpu
from jax.experimental.pallas import tpu_sc as plsc

N = 10000
E = 160000
D = 256
C = 256
K = 16

NSC = 2
NSUB = 16
CH = C // NSC
EPS = E // NSUB
CHUNK = 128
NCHUNK = -(-EPS // CHUNK)
EPP = NCHUNK * CHUNK
TRASH = N
AGG_ROWS = 10240
RPS = AGG_ROWS // NSUB


def _h_body(x_ref, w1_ref, h_ref):
    h_ref[...] = jnp.dot(x_ref[...], w1_ref[...],
                         preferred_element_type=jnp.float32)


def _h_split(x, W1, rb=400):
    nb = N // rb
    return pl.pallas_call(
        _h_body,
        grid=(nb, NSC),
        in_specs=[
            pl.BlockSpec((rb, D), lambda i, c: (i, 0)),
            pl.BlockSpec((D, CH), lambda i, c: (0, c)),
        ],
        out_specs=pl.BlockSpec((rb, CH), lambda i, c: (c * nb + i, 0)),
        out_shape=jax.ShapeDtypeStruct((NSC * N, CH), jnp.float32),
    )(x, W1)


def _sc_body(h_hbm, src_hbm, dst_hbm, out_hbm, agg_sh, dst_v,
             sa, sb, rows0, rows1, sem_ia, sem_ib, semg0, semg1, sems0, sems1):
    c = lax.axis_index("c")
    s = lax.axis_index("s")
    sbase = (c * NSUB + s) * EPP

    def src_load(j, sref, sem):
        pltpu.async_copy(src_hbm.at[pl.ds(sbase + j * CHUNK, CHUNK)], sref, sem)

    def src_wait(j, sref, sem):
        pltpu.make_async_copy(
            src_hbm.at[pl.ds(sbase + j * CHUNK, CHUNK)], sref, sem).wait()

    def gather_wait(sref, rows, sem):
        pltpu.make_async_copy(h_hbm.at[sref], rows, sem).wait()

    def scat_start(j, rows, sem):
        pltpu.async_copy(rows, agg_sh.at[dst_v.at[j]], sem, add=True)

    def scat_wait(j, rows, sem):
        pltpu.make_async_copy(rows, agg_sh.at[dst_v.at[j]], sem).wait()

    pltpu.sync_copy(dst_hbm.at[s], dst_v)

    @pl.loop(0, CHUNK)
    def _(i):
        for k in range(CH // 16):
            rows0[i, pl.ds(k * 16, 16)] = jnp.zeros((16,), jnp.float32)

    @pl.loop(0, RPS // CHUNK)
    def _(r):
        pltpu.sync_copy(rows0, agg_sh.at[pl.ds(s * RPS + r * CHUNK, CHUNK)])

    plsc.subcore_barrier()

    src_load(0, sa, sem_ia)
    src_load(1, sb, sem_ib)
    src_wait(0, sa, sem_ia)
    pltpu.async_copy(h_hbm.at[sa], rows0, semg0)

    @pl.loop(0, (NCHUNK - 3) // 2)
    def _(p):
        j = 2 * p
        src_wait(j + 1, sb, sem_ib)

        @pl.when(p > 0)
        def _():
            scat_wait(j - 1, rows1, sems1)

        pltpu.async_copy(h_hbm.at[sb], rows1, semg1)
        gather_wait(sa, rows0, semg0)
        scat_start(j, rows0, sems0)
        src_load(j + 2, sa, sem_ia)
        src_wait(j + 2, sa, sem_ia)
        scat_wait(j, rows0, sems0)
        pltpu.async_copy(h_hbm.at[sa], rows0, semg0)
        gather_wait(sb, rows1, semg1)
        scat_start(j + 1, rows1, sems1)
        src_load(j + 3, sb, sem_ib)

    J = NCHUNK - 3
    src_wait(J + 1, sb, sem_ib)
    scat_wait(J - 1, rows1, sems1)
    pltpu.async_copy(h_hbm.at[sb], rows1, semg1)
    gather_wait(sa, rows0, semg0)
    scat_start(J, rows0, sems0)
    src_load(J + 2, sa, sem_ia)
    src_wait(J + 2, sa, sem_ia)
    scat_wait(J, rows0, sems0)
    pltpu.async_copy(h_hbm.at[sa], rows0, semg0)
    gather_wait(sb, rows1, semg1)
    scat_start(J + 1, rows1, sems1)
    gather_wait(sa, rows0, semg0)
    scat_wait(J + 1, rows1, sems1)
    pltpu.sync_copy(rows0, agg_sh.at[dst_v.at[J + 2]], add=True)

    plsc.subcore_barrier()

    wbase = s * 624
    pltpu.sync_copy(agg_sh.at[pl.ds(wbase, 624)],
                    out_hbm.at[pl.ds(c * N + wbase, 624)])

    @pl.when(s == NSUB - 1)
    def _():
        pltpu.sync_copy(agg_sh.at[pl.ds(624 * NSUB, N - 624 * NSUB)],
                        out_hbm.at[pl.ds(c * N + 624 * NSUB, N - 624 * NSUB)])


@functools.partial(
    pl.kernel,
    out_type=jax.ShapeDtypeStruct((NSC * N, CH), jnp.float32),
    mesh=plsc.VectorSubcoreMesh(core_axis_name="c", subcore_axis_name="s"),
    scratch_types=[
        pltpu.VMEM_SHARED((AGG_ROWS, CH), jnp.float32),
        pltpu.VMEM((NCHUNK, CHUNK), jnp.int32),
        pltpu.VMEM((CHUNK,), jnp.int32),
        pltpu.VMEM((CHUNK,), jnp.int32),
        pltpu.VMEM((CHUNK, CH), jnp.float32),
        pltpu.VMEM((CHUNK, CH), jnp.float32),
        pltpu.SemaphoreType.DMA,
        pltpu.SemaphoreType.DMA,
        pltpu.SemaphoreType.DMA,
        pltpu.SemaphoreType.DMA,
        pltpu.SemaphoreType.DMA,
        pltpu.SemaphoreType.DMA,
    ],
)
def _sc_scatter(h_hbm, src_hbm, dst_hbm, out_hbm, agg_sh, dst_v,
                sa, sb, rows0, rows1,
                sem_ia, sem_ib, semg0, semg1, sems0, sems1):
    _sc_body(h_hbm, src_hbm, dst_hbm, out_hbm, agg_sh, dst_v,
             sa, sb, rows0, rows1,
             sem_ia, sem_ib, semg0, semg1, sems0, sems1)


def _out_body(agg_ref, x_ref, w2_ref, b_ref, ws_ref, bs_ref, o_ref):
    c = pl.program_id(1)
    t = agg_ref[...] + jnp.dot(x_ref[...], w2_ref[...],
                               preferred_element_type=jnp.float32) + b_ref[0]
    t = jnp.where(t > 0, t, jnp.exp(jnp.minimum(t, 0.0)) - 1.0)
    part = jnp.dot(t, ws_ref[0], preferred_element_type=jnp.float32)

    @pl.when(c == 0)
    def _():
        o_ref[...] = part

    @pl.when(c == 1)
    def _():
        z = o_ref[...] + part + bs_ref[...]
        m = jnp.max(z, axis=-1, keepdims=True)
        e = jnp.exp(z - m)
        o_ref[...] = e / jnp.sum(e, axis=-1, keepdims=True)


def _finish(agg_split, x, W2, b, Ws, bs, rb=400):
    nb = N // rb
    return pl.pallas_call(
        _out_body,
        grid=(nb, NSC),
        in_specs=[
            pl.BlockSpec((rb, CH), lambda i, c: (c * nb + i, 0)),
            pl.BlockSpec((rb, D), lambda i, c: (i, 0)),
            pl.BlockSpec((D, CH), lambda i, c: (0, c)),
            pl.BlockSpec((1, 1, CH), lambda i, c: (c, 0, 0)),
            pl.BlockSpec((1, CH, K), lambda i, c: (c, 0, 0)),
            pl.BlockSpec((1, K), lambda i, c: (0, 0)),
        ],
        out_specs=pl.BlockSpec((rb, K), lambda i, c: (i, 0)),
        out_shape=jax.ShapeDtypeStruct((N, K), jnp.float32),
    )(agg_split, x, W2, b.reshape(NSC, 1, CH), Ws.reshape(NSC, CH, K),
      bs.reshape(1, K))


def kernel(x, edge_index, W1, W2, b, Ws, bs):
    src = edge_index[0]
    dst = edge_index[1]
    srcp = jnp.pad(src.reshape(NSUB, EPS), ((0, 0), (0, EPP - EPS)))
    dstp = jnp.pad(dst.reshape(NSUB, EPS), ((0, 0), (0, EPP - EPS)),
                   constant_values=TRASH)
    srcf = jnp.concatenate([srcp, srcp + N]).reshape(-1)
    dstf = dstp.reshape(NSUB, NCHUNK, CHUNK)

    h_split = _h_split(x, W1)
    agg_split = _sc_scatter(h_split, srcf, dstf)
    return _finish(agg_split, x, W2, b, Ws, bs)


# speedup vs baseline: 5.4621x; 1.1507x over previous
import functools

import jax
import jax.numpy as jnp
from jax import lax
from jax.experimental import pallas as pl
from jax.experimental.pallas import tpu as pltpu
from jax.experimental.pallas import tpu_sc as plsc

N = 10000
E = 160000
D = 256
C = 256
K = 16

NSC = 2
NSUB = 16
CH = C // NSC
EPS = E // NSUB
CHUNK = 128
NCHUNK = -(-EPS // CHUNK)
EPP = NCHUNK * CHUNK
TRASH = N
AGG_ROWS = 10240
RPS = AGG_ROWS // NSUB


def _h_body(x_ref, w1_ref, h_ref):
    hb = jnp.dot(x_ref[...], w1_ref[...], preferred_element_type=jnp.float32)
    h_ref[0] = hb[:, :CH]
    h_ref[1] = hb[:, CH:]


def _h_split(x, W1, rb=400):
    nb = N // rb
    return pl.pallas_call(
        _h_body,
        grid=(nb,),
        in_specs=[
            pl.BlockSpec((rb, D), lambda i: (i, 0)),
            pl.BlockSpec((D, C), lambda i: (0, 0)),
        ],
        out_specs=pl.BlockSpec((NSC, rb, CH), lambda i: (0, i, 0)),
        out_shape=jax.ShapeDtypeStruct((NSC, N, CH), jnp.float32),
    )(x, W1).reshape(NSC * N, CH)


def _sc_body(h_hbm, src_hbm, dst_hbm, out_hbm, agg_sh, dst_v,
             sa, sb, rows0, rows1, sem_ia, sem_ib, semg0, semg1, sems0, sems1):
    c = lax.axis_index("c")
    s = lax.axis_index("s")
    sbase = (c * NSUB + s) * EPP

    def src_load(j, sref, sem):
        pltpu.async_copy(src_hbm.at[pl.ds(sbase + j * CHUNK, CHUNK)], sref, sem)

    def src_wait(j, sref, sem):
        pltpu.make_async_copy(
            src_hbm.at[pl.ds(sbase + j * CHUNK, CHUNK)], sref, sem).wait()

    def gather_wait(sref, rows, sem):
        pltpu.make_async_copy(h_hbm.at[sref], rows, sem).wait()

    def scat_start(j, rows, sem):
        pltpu.async_copy(rows, agg_sh.at[dst_v.at[j]], sem, add=True)

    def scat_wait(j, rows, sem):
        pltpu.make_async_copy(rows, agg_sh.at[dst_v.at[j]], sem).wait()

    pltpu.sync_copy(dst_hbm.at[s], dst_v)

    @pl.loop(0, CHUNK)
    def _(i):
        for k in range(CH // 16):
            rows0[i, pl.ds(k * 16, 16)] = jnp.zeros((16,), jnp.float32)

    @pl.loop(0, RPS // CHUNK)
    def _(r):
        pltpu.sync_copy(rows0, agg_sh.at[pl.ds(s * RPS + r * CHUNK, CHUNK)])

    plsc.subcore_barrier()

    src_load(0, sa, sem_ia)
    src_load(1, sb, sem_ib)
    src_wait(0, sa, sem_ia)
    pltpu.async_copy(h_hbm.at[sa], rows0, semg0)

    @pl.loop(0, (NCHUNK - 3) // 2)
    def _(p):
        j = 2 * p
        src_wait(j + 1, sb, sem_ib)

        @pl.when(p > 0)
        def _():
            scat_wait(j - 1, rows1, sems1)

        pltpu.async_copy(h_hbm.at[sb], rows1, semg1)
        gather_wait(sa, rows0, semg0)
        scat_start(j, rows0, sems0)
        src_load(j + 2, sa, sem_ia)
        src_wait(j + 2, sa, sem_ia)
        scat_wait(j, rows0, sems0)
        pltpu.async_copy(h_hbm.at[sa], rows0, semg0)
        gather_wait(sb, rows1, semg1)
        scat_start(j + 1, rows1, sems1)
        src_load(j + 3, sb, sem_ib)

    J = NCHUNK - 3
    src_wait(J + 1, sb, sem_ib)
    scat_wait(J - 1, rows1, sems1)
    pltpu.async_copy(h_hbm.at[sb], rows1, semg1)
    gather_wait(sa, rows0, semg0)
    scat_start(J, rows0, sems0)
    src_load(J + 2, sa, sem_ia)
    src_wait(J + 2, sa, sem_ia)
    scat_wait(J, rows0, sems0)
    pltpu.async_copy(h_hbm.at[sa], rows0, semg0)
    gather_wait(sb, rows1, semg1)
    scat_start(J + 1, rows1, sems1)
    gather_wait(sa, rows0, semg0)
    scat_wait(J + 1, rows1, sems1)
    pltpu.sync_copy(rows0, agg_sh.at[dst_v.at[J + 2]], add=True)

    plsc.subcore_barrier()

    wbase = s * 624
    pltpu.sync_copy(agg_sh.at[pl.ds(wbase, 624)],
                    out_hbm.at[pl.ds(c * N + wbase, 624)])

    @pl.when(s == NSUB - 1)
    def _():
        pltpu.sync_copy(agg_sh.at[pl.ds(624 * NSUB, N - 624 * NSUB)],
                        out_hbm.at[pl.ds(c * N + 624 * NSUB, N - 624 * NSUB)])


@functools.partial(
    pl.kernel,
    out_type=jax.ShapeDtypeStruct((NSC * N, CH), jnp.float32),
    mesh=plsc.VectorSubcoreMesh(core_axis_name="c", subcore_axis_name="s"),
    scratch_types=[
        pltpu.VMEM_SHARED((AGG_ROWS, CH), jnp.float32),
        pltpu.VMEM((NCHUNK, CHUNK), jnp.int32),
        pltpu.VMEM((CHUNK,), jnp.int32),
        pltpu.VMEM((CHUNK,), jnp.int32),
        pltpu.VMEM((CHUNK, CH), jnp.float32),
        pltpu.VMEM((CHUNK, CH), jnp.float32),
        pltpu.SemaphoreType.DMA,
        pltpu.SemaphoreType.DMA,
        pltpu.SemaphoreType.DMA,
        pltpu.SemaphoreType.DMA,
        pltpu.SemaphoreType.DMA,
        pltpu.SemaphoreType.DMA,
    ],
)
def _sc_scatter(h_hbm, src_hbm, dst_hbm, out_hbm, agg_sh, dst_v,
                sa, sb, rows0, rows1,
                sem_ia, sem_ib, semg0, semg1, sems0, sems1):
    _sc_body(h_hbm, src_hbm, dst_hbm, out_hbm, agg_sh, dst_v,
             sa, sb, rows0, rows1,
             sem_ia, sem_ib, semg0, semg1, sems0, sems1)


def _out_body(agg_ref, x_ref, w2_ref, b_ref, ws_ref, bs_ref, o_ref):
    agg = jnp.concatenate([agg_ref[0], agg_ref[1]], axis=-1)
    t = agg + jnp.dot(x_ref[...], w2_ref[...],
                      preferred_element_type=jnp.float32) + b_ref[...]
    t = jnp.where(t > 0, t, jnp.exp(jnp.minimum(t, 0.0)) - 1.0)
    z = jnp.dot(t, ws_ref[...], preferred_element_type=jnp.float32) + bs_ref[...]
    m = jnp.max(z, axis=-1, keepdims=True)
    e = jnp.exp(z - m)
    o_ref[...] = e / jnp.sum(e, axis=-1, keepdims=True)


def _finish(agg_split, x, W2, b, Ws, bs, rb=400):
    nb = N // rb
    return pl.pallas_call(
        _out_body,
        grid=(nb,),
        in_specs=[
            pl.BlockSpec((NSC, rb, CH), lambda i: (0, i, 0)),
            pl.BlockSpec((rb, D), lambda i: (i, 0)),
            pl.BlockSpec((D, C), lambda i: (0, 0)),
            pl.BlockSpec((1, C), lambda i: (0, 0)),
            pl.BlockSpec((C, K), lambda i: (0, 0)),
            pl.BlockSpec((1, K), lambda i: (0, 0)),
        ],
        out_specs=pl.BlockSpec((rb, K), lambda i: (i, 0)),
        out_shape=jax.ShapeDtypeStruct((N, K), jnp.float32),
    )(agg_split.reshape(NSC, N, CH), x, W2, b.reshape(1, C), Ws,
      bs.reshape(1, K))


def kernel(x, edge_index, W1, W2, b, Ws, bs):
    src = edge_index[0]
    dst = edge_index[1]
    srcp = jnp.pad(src.reshape(NSUB, EPS), ((0, 0), (0, EPP - EPS)))
    dstp = jnp.pad(dst.reshape(NSUB, EPS), ((0, 0), (0, EPP - EPS)),
                   constant_values=TRASH)
    srcf = jnp.concatenate([srcp, srcp + N]).reshape(-1)
    dstf = dstp.reshape(NSUB, NCHUNK, CHUNK)

    h_split = _h_split(x, W1)
    agg_split = _sc_scatter(h_split, srcf, dstf)
    return _finish(agg_split, x, W2, b, Ws, bs)


# speedup vs baseline: 5.4682x; 1.0011x over previous
import functools

import jax
import jax.numpy as jnp
from jax import lax
from jax.experimental import pallas as pl
from jax.experimental.pallas import tpu as pltpu
from jax.experimental.pallas import tpu_sc as plsc

N = 10000
E = 160000
D = 256
C = 256
K = 16

NSC = 2
NSUB = 16
CH = C // NSC
EPS = E // NSUB
CHUNK = 128
NCHUNK = -(-EPS // CHUNK)
EPP = NCHUNK * CHUNK
TRASH = N
AGG_ROWS = 10240
RPS = AGG_ROWS // NSUB


def _h_body(x_ref, w1_ref, h_ref):
    hb = jnp.dot(x_ref[...], w1_ref[...], preferred_element_type=jnp.float32)
    h_ref[0] = hb[:, :CH]
    h_ref[1] = hb[:, CH:]


def _h_split(x, W1, rb=400):
    nb = N // rb
    return pl.pallas_call(
        _h_body,
        grid=(nb,),
        in_specs=[
            pl.BlockSpec((rb, D), lambda i: (i, 0)),
            pl.BlockSpec((D, C), lambda i: (0, 0)),
        ],
        out_specs=pl.BlockSpec((NSC, rb, CH), lambda i: (0, i, 0)),
        out_shape=jax.ShapeDtypeStruct((NSC, N, CH), jnp.float32),
    )(x, W1).reshape(NSC * N, CH)


def _sc_body(h_hbm, src_hbm, dst_hbm, out_hbm, agg_sh, dst_v,
             sa, sb, rows0, rows1, sem_ia, sem_ib, semg0, semg1, sems0, sems1):
    c = lax.axis_index("c")
    s = lax.axis_index("s")
    sbase = (c * NSUB + s) * EPP

    def src_load(j, sref, sem):
        pltpu.async_copy(src_hbm.at[pl.ds(sbase + j * CHUNK, CHUNK)], sref, sem)

    def src_wait(j, sref, sem):
        pltpu.make_async_copy(
            src_hbm.at[pl.ds(sbase + j * CHUNK, CHUNK)], sref, sem).wait()

    def gather_wait(sref, rows, sem):
        pltpu.make_async_copy(h_hbm.at[sref], rows, sem).wait()

    def scat_start(j, rows, sem):
        pltpu.async_copy(rows, agg_sh.at[dst_v.at[j]], sem, add=True)

    def scat_wait(j, rows, sem):
        pltpu.make_async_copy(rows, agg_sh.at[dst_v.at[j]], sem).wait()

    pltpu.sync_copy(dst_hbm.at[s], dst_v)

    @pl.loop(0, CHUNK)
    def _(i):
        for k in range(CH // 16):
            rows0[i, pl.ds(k * 16, 16)] = jnp.zeros((16,), jnp.float32)

    @pl.loop(0, RPS // CHUNK)
    def _(r):
        pltpu.sync_copy(rows0, agg_sh.at[pl.ds(s * RPS + r * CHUNK, CHUNK)])

    plsc.subcore_barrier()

    src_load(0, sa, sem_ia)
    src_load(1, sb, sem_ib)
    src_wait(0, sa, sem_ia)
    pltpu.async_copy(h_hbm.at[sa], rows0, semg0)

    @pl.loop(0, (NCHUNK - 3) // 2)
    def _(p):
        j = 2 * p
        src_wait(j + 1, sb, sem_ib)

        @pl.when(p > 0)
        def _():
            scat_wait(j - 1, rows1, sems1)

        pltpu.async_copy(h_hbm.at[sb], rows1, semg1)
        gather_wait(sa, rows0, semg0)
        scat_start(j, rows0, sems0)
        src_load(j + 2, sa, sem_ia)
        src_wait(j + 2, sa, sem_ia)
        scat_wait(j, rows0, sems0)
        pltpu.async_copy(h_hbm.at[sa], rows0, semg0)
        gather_wait(sb, rows1, semg1)
        scat_start(j + 1, rows1, sems1)
        src_load(j + 3, sb, sem_ib)

    J = NCHUNK - 3
    src_wait(J + 1, sb, sem_ib)
    scat_wait(J - 1, rows1, sems1)
    pltpu.async_copy(h_hbm.at[sb], rows1, semg1)
    gather_wait(sa, rows0, semg0)
    scat_start(J, rows0, sems0)
    src_load(J + 2, sa, sem_ia)
    src_wait(J + 2, sa, sem_ia)
    scat_wait(J, rows0, sems0)
    pltpu.async_copy(h_hbm.at[sa], rows0, semg0)
    gather_wait(sb, rows1, semg1)
    scat_start(J + 1, rows1, sems1)
    gather_wait(sa, rows0, semg0)
    scat_wait(J + 1, rows1, sems1)
    pltpu.sync_copy(rows0, agg_sh.at[dst_v.at[J + 2]], add=True)

    plsc.subcore_barrier()

    wbase = s * 624
    pltpu.sync_copy(agg_sh.at[pl.ds(wbase, 624)],
                    out_hbm.at[pl.ds(c * N + wbase, 624)])

    @pl.when(s == NSUB - 1)
    def _():
        pltpu.sync_copy(agg_sh.at[pl.ds(624 * NSUB, N - 624 * NSUB)],
                        out_hbm.at[pl.ds(c * N + 624 * NSUB, N - 624 * NSUB)])


@functools.partial(
    pl.kernel,
    out_type=jax.ShapeDtypeStruct((NSC * N, CH), jnp.float32),
    mesh=plsc.VectorSubcoreMesh(core_axis_name="c", subcore_axis_name="s"),
    scratch_types=[
        pltpu.VMEM_SHARED((AGG_ROWS, CH), jnp.float32),
        pltpu.VMEM((NCHUNK, CHUNK), jnp.int32),
        pltpu.VMEM((CHUNK,), jnp.int32),
        pltpu.VMEM((CHUNK,), jnp.int32),
        pltpu.VMEM((CHUNK, CH), jnp.float32),
        pltpu.VMEM((CHUNK, CH), jnp.float32),
        pltpu.SemaphoreType.DMA,
        pltpu.SemaphoreType.DMA,
        pltpu.SemaphoreType.DMA,
        pltpu.SemaphoreType.DMA,
        pltpu.SemaphoreType.DMA,
        pltpu.SemaphoreType.DMA,
    ],
)
def _sc_scatter(h_hbm, src_hbm, dst_hbm, out_hbm, agg_sh, dst_v,
                sa, sb, rows0, rows1,
                sem_ia, sem_ib, semg0, semg1, sems0, sems1):
    _sc_body(h_hbm, src_hbm, dst_hbm, out_hbm, agg_sh, dst_v,
             sa, sb, rows0, rows1,
             sem_ia, sem_ib, semg0, semg1, sems0, sems1)


def _skip_body(x_ref, w2_ref, s_ref):
    s_ref[...] = jnp.dot(x_ref[...], w2_ref[...],
                         preferred_element_type=jnp.float32)


def _skip(x, W2, rb=400):
    nb = N // rb
    return pl.pallas_call(
        _skip_body,
        grid=(nb,),
        in_specs=[
            pl.BlockSpec((rb, D), lambda i: (i, 0)),
            pl.BlockSpec((D, C), lambda i: (0, 0)),
        ],
        out_specs=pl.BlockSpec((rb, C), lambda i: (i, 0)),
        out_shape=jax.ShapeDtypeStruct((N, C), jnp.float32),
    )(x, W2)


def _out_body(agg_ref, sk_ref, b_ref, ws_ref, bs_ref, o_ref):
    agg = jnp.concatenate([agg_ref[0], agg_ref[1]], axis=-1)
    t = agg + sk_ref[...] + b_ref[...]
    t = jnp.where(t > 0, t, jnp.exp(jnp.minimum(t, 0.0)) - 1.0)
    z = jnp.dot(t, ws_ref[...], preferred_element_type=jnp.float32) + bs_ref[...]
    m = jnp.max(z, axis=-1, keepdims=True)
    e = jnp.exp(z - m)
    o_ref[...] = e / jnp.sum(e, axis=-1, keepdims=True)


def _finish(agg_split, skip, b, Ws, bs, rb=400):
    nb = N // rb
    return pl.pallas_call(
        _out_body,
        grid=(nb,),
        in_specs=[
            pl.BlockSpec((NSC, rb, CH), lambda i: (0, i, 0)),
            pl.BlockSpec((rb, C), lambda i: (i, 0)),
            pl.BlockSpec((1, C), lambda i: (0, 0)),
            pl.BlockSpec((C, K), lambda i: (0, 0)),
            pl.BlockSpec((1, K), lambda i: (0, 0)),
        ],
        out_specs=pl.BlockSpec((rb, K), lambda i: (i, 0)),
        out_shape=jax.ShapeDtypeStruct((N, K), jnp.float32),
    )(agg_split.reshape(NSC, N, CH), skip, b.reshape(1, C), Ws,
      bs.reshape(1, K))


def kernel(x, edge_index, W1, W2, b, Ws, bs):
    src = edge_index[0]
    dst = edge_index[1]
    srcp = jnp.pad(src.reshape(NSUB, EPS), ((0, 0), (0, EPP - EPS)))
    dstp = jnp.pad(dst.reshape(NSUB, EPS), ((0, 0), (0, EPP - EPS)),
                   constant_values=TRASH)
    srcf = jnp.concatenate([srcp, srcp + N]).reshape(-1)
    dstf = dstp.reshape(NSUB, NCHUNK, CHUNK)

    h_split = _h_split(x, W1)
    agg_split = _sc_scatter(h_split, srcf, dstf)
    skip = _skip(x, W2)
    return _finish(agg_split, skip, b, Ws, bs)


# speedup vs baseline: 6.8402x; 1.2509x over previous
import functools

import jax
import jax.numpy as jnp
from jax import lax
from jax.experimental import pallas as pl
from jax.experimental.pallas import tpu as pltpu
from jax.experimental.pallas import tpu_sc as plsc

N = 10000
E = 160000
D = 256
C = 256
K = 16

NSC = 2
NSUB = 16
CH = C // NSC
EPS = E // NSUB
CHUNK = 80
NCHUNK = EPS // CHUNK
AGG_ROWS = 10240
RPS = AGG_ROWS // NSUB


def _h_body(x_ref, w1_ref, h_ref):
    hb = jnp.dot(x_ref[...], w1_ref[...], preferred_element_type=jnp.float32)
    h_ref[0] = hb[:, :CH]
    h_ref[1] = hb[:, CH:]


def _h_split(x, W1, rb=400):
    nb = N // rb
    return pl.pallas_call(
        _h_body,
        grid=(nb,),
        in_specs=[
            pl.BlockSpec((rb, D), lambda i: (i, 0)),
            pl.BlockSpec((D, C), lambda i: (0, 0)),
        ],
        out_specs=pl.BlockSpec((NSC, rb, CH), lambda i: (0, i, 0)),
        out_shape=jax.ShapeDtypeStruct((NSC, N, CH), jnp.float32),
    )(x, W1).reshape(NSC * N, CH)


def _sc_body(h_hbm, src_hbm, dst_hbm, out_hbm, agg_sh, dst_v,
             sa, sb, rows0, rows1, sem_ia, sem_ib, semg0, semg1, sems0, sems1):
    c = lax.axis_index("c")
    s = lax.axis_index("s")
    sbase = s * EPS
    off = c * N

    def src_load(j, sref, sem):
        pltpu.async_copy(src_hbm.at[pl.ds(sbase + j * CHUNK, CHUNK)], sref, sem)

    def src_wait(j, sref, sem):
        pltpu.make_async_copy(
            src_hbm.at[pl.ds(sbase + j * CHUNK, CHUNK)], sref, sem).wait()
        for k in range(CHUNK // 16):
            sref[pl.ds(k * 16, 16)] += off

    def gather_wait(sref, rows, sem):
        pltpu.make_async_copy(h_hbm.at[sref], rows, sem).wait()

    def scat_start(j, rows, sem):
        pltpu.async_copy(rows, agg_sh.at[dst_v.at[j]], sem, add=True)

    def scat_wait(j, rows, sem):
        pltpu.make_async_copy(rows, agg_sh.at[dst_v.at[j]], sem).wait()

    pltpu.sync_copy(dst_hbm.at[s], dst_v)

    @pl.loop(0, CHUNK)
    def _(i):
        for k in range(CH // 16):
            rows0[i, pl.ds(k * 16, 16)] = jnp.zeros((16,), jnp.float32)

    @pl.loop(0, RPS // CHUNK)
    def _(r):
        pltpu.sync_copy(rows0, agg_sh.at[pl.ds(s * RPS + r * CHUNK, CHUNK)])

    plsc.subcore_barrier()

    src_load(0, sa, sem_ia)
    src_load(1, sb, sem_ib)
    src_wait(0, sa, sem_ia)
    pltpu.async_copy(h_hbm.at[sa], rows0, semg0)

    @pl.loop(0, (NCHUNK - 3) // 2)
    def _(p):
        j = 2 * p
        src_wait(j + 1, sb, sem_ib)

        @pl.when(p > 0)
        def _():
            scat_wait(j - 1, rows1, sems1)

        pltpu.async_copy(h_hbm.at[sb], rows1, semg1)
        gather_wait(sa, rows0, semg0)
        scat_start(j, rows0, sems0)
        src_load(j + 2, sa, sem_ia)
        src_wait(j + 2, sa, sem_ia)
        scat_wait(j, rows0, sems0)
        pltpu.async_copy(h_hbm.at[sa], rows0, semg0)
        gather_wait(sb, rows1, semg1)
        scat_start(j + 1, rows1, sems1)
        src_load(j + 3, sb, sem_ib)

    J = NCHUNK - 3
    src_wait(J + 1, sb, sem_ib)
    scat_wait(J - 1, rows1, sems1)
    pltpu.async_copy(h_hbm.at[sb], rows1, semg1)
    gather_wait(sa, rows0, semg0)
    scat_start(J, rows0, sems0)
    src_load(J + 2, sa, sem_ia)
    src_wait(J + 2, sa, sem_ia)
    scat_wait(J, rows0, sems0)
    pltpu.async_copy(h_hbm.at[sa], rows0, semg0)
    gather_wait(sb, rows1, semg1)
    scat_start(J + 1, rows1, sems1)
    gather_wait(sa, rows0, semg0)
    scat_wait(J + 1, rows1, sems1)
    pltpu.sync_copy(rows0, agg_sh.at[dst_v.at[J + 2]], add=True)

    plsc.subcore_barrier()

    wbase = s * 624
    pltpu.sync_copy(agg_sh.at[pl.ds(wbase, 624)],
                    out_hbm.at[pl.ds(c * N + wbase, 624)])

    @pl.when(s == NSUB - 1)
    def _():
        pltpu.sync_copy(agg_sh.at[pl.ds(624 * NSUB, N - 624 * NSUB)],
                        out_hbm.at[pl.ds(c * N + 624 * NSUB, N - 624 * NSUB)])


@functools.partial(
    pl.kernel,
    out_type=jax.ShapeDtypeStruct((NSC * N, CH), jnp.float32),
    mesh=plsc.VectorSubcoreMesh(core_axis_name="c", subcore_axis_name="s"),
    scratch_types=[
        pltpu.VMEM_SHARED((AGG_ROWS, CH), jnp.float32),
        pltpu.VMEM((NCHUNK, CHUNK), jnp.int32),
        pltpu.VMEM((CHUNK,), jnp.int32),
        pltpu.VMEM((CHUNK,), jnp.int32),
        pltpu.VMEM((CHUNK, CH), jnp.float32),
        pltpu.VMEM((CHUNK, CH), jnp.float32),
        pltpu.SemaphoreType.DMA,
        pltpu.SemaphoreType.DMA,
        pltpu.SemaphoreType.DMA,
        pltpu.SemaphoreType.DMA,
        pltpu.SemaphoreType.DMA,
        pltpu.SemaphoreType.DMA,
    ],
)
def _sc_scatter(h_hbm, src_hbm, dst_hbm, out_hbm, agg_sh, dst_v,
                sa, sb, rows0, rows1,
                sem_ia, sem_ib, semg0, semg1, sems0, sems1):
    _sc_body(h_hbm, src_hbm, dst_hbm, out_hbm, agg_sh, dst_v,
             sa, sb, rows0, rows1,
             sem_ia, sem_ib, semg0, semg1, sems0, sems1)


def _skip_body(x_ref, w2_ref, s_ref):
    s_ref[...] = jnp.dot(x_ref[...], w2_ref[...],
                         preferred_element_type=jnp.float32)


def _skip(x, W2, rb=400):
    nb = N // rb
    return pl.pallas_call(
        _skip_body,
        grid=(nb,),
        in_specs=[
            pl.BlockSpec((rb, D), lambda i: (i, 0)),
            pl.BlockSpec((D, C), lambda i: (0, 0)),
        ],
        out_specs=pl.BlockSpec((rb, C), lambda i: (i, 0)),
        out_shape=jax.ShapeDtypeStruct((N, C), jnp.float32),
    )(x, W2)


def _out_body(agg_ref, sk_ref, b_ref, ws_ref, bs_ref, o_ref):
    agg = jnp.concatenate([agg_ref[0], agg_ref[1]], axis=-1)
    t = agg + sk_ref[...] + b_ref[...]
    t = jnp.where(t > 0, t, jnp.exp(jnp.minimum(t, 0.0)) - 1.0)
    z = jnp.dot(t, ws_ref[...], preferred_element_type=jnp.float32) + bs_ref[...]
    m = jnp.max(z, axis=-1, keepdims=True)
    e = jnp.exp(z - m)
    o_ref[...] = e / jnp.sum(e, axis=-1, keepdims=True)


def _finish(agg_split, skip, b, Ws, bs, rb=400):
    nb = N // rb
    return pl.pallas_call(
        _out_body,
        grid=(nb,),
        in_specs=[
            pl.BlockSpec((NSC, rb, CH), lambda i: (0, i, 0)),
            pl.BlockSpec((rb, C), lambda i: (i, 0)),
            pl.BlockSpec((1, C), lambda i: (0, 0)),
            pl.BlockSpec((C, K), lambda i: (0, 0)),
            pl.BlockSpec((1, K), lambda i: (0, 0)),
        ],
        out_specs=pl.BlockSpec((rb, K), lambda i: (i, 0)),
        out_shape=jax.ShapeDtypeStruct((N, K), jnp.float32),
    )(agg_split.reshape(NSC, N, CH), skip, b.reshape(1, C), Ws,
      bs.reshape(1, K))


def kernel(x, edge_index, W1, W2, b, Ws, bs):
    src = edge_index[0]
    dst = edge_index[1]
    srcf = src
    dstf = dst.reshape(NSUB, NCHUNK, CHUNK)

    h_split = _h_split(x, W1)
    agg_split = _sc_scatter(h_split, srcf, dstf)
    skip = _skip(x, W2)
    return _finish(agg_split, skip, b, Ws, bs)


# speedup vs baseline: 7.4928x; 1.0954x over previous
import functools

import jax
import jax.numpy as jnp
from jax import lax
from jax.experimental import pallas as pl
from jax.experimental.pallas import tpu as pltpu
from jax.experimental.pallas import tpu_sc as plsc

N = 10000
E = 160000
D = 256
C = 256
K = 16

NSC = 2
NSUB = 16
CH = C // NSC
EPS = E // NSUB
CHUNK = 80
NCHUNK = EPS // CHUNK
AGG_ROWS = 10240
RPS = AGG_ROWS // NSUB


def _h_body(x_ref, w1_ref, h_ref):
    hb = jnp.dot(x_ref[...], w1_ref[...], preferred_element_type=jnp.float32)
    h_ref[0] = hb[:, :CH]
    h_ref[1] = hb[:, CH:]


def _h_split(x, W1, rb=1000):
    nb = N // rb
    return pl.pallas_call(
        _h_body,
        grid=(nb,),
        in_specs=[
            pl.BlockSpec((rb, D), lambda i: (i, 0)),
            pl.BlockSpec((D, C), lambda i: (0, 0)),
        ],
        out_specs=pl.BlockSpec((NSC, rb, CH), lambda i: (0, i, 0)),
        out_shape=jax.ShapeDtypeStruct((NSC, N, CH), jnp.float32),
    )(x, W1).reshape(NSC * N, CH)


def _sc_body(ei_hbm, h_hbm, out_hbm, agg_sh, dst_v,
             sa, sb, rows0, rows1, sem_ia, sem_ib, semg0, semg1, sems0, sems1):
    c = lax.axis_index("c")
    s = lax.axis_index("s")
    off = c * N

    def src_load(j, sref, sem):
        pltpu.async_copy(ei_hbm.at[0, s, j], sref, sem)

    def src_wait(j, sref, sem):
        pltpu.make_async_copy(ei_hbm.at[0, s, j], sref, sem).wait()
        for k in range(CHUNK // 16):
            sref[0, pl.ds(k * 16, 16)] += off

    def gather_wait(sref, rows, sem):
        pltpu.make_async_copy(h_hbm.at[sref.at[0]], rows, sem).wait()

    def scat_start(j, rows, sem):
        pltpu.async_copy(rows, agg_sh.at[dst_v.at[j, 0]], sem, add=True)

    def scat_wait(j, rows, sem):
        pltpu.make_async_copy(rows, agg_sh.at[dst_v.at[j, 0]], sem).wait()

    pltpu.sync_copy(ei_hbm.at[1, s], dst_v)

    @pl.loop(0, CHUNK)
    def _(i):
        for k in range(CH // 16):
            rows0[i, pl.ds(k * 16, 16)] = jnp.zeros((16,), jnp.float32)

    @pl.loop(0, RPS // CHUNK)
    def _(r):
        pltpu.sync_copy(rows0, agg_sh.at[pl.ds(s * RPS + r * CHUNK, CHUNK)])

    plsc.subcore_barrier()

    src_load(0, sa, sem_ia)
    src_load(1, sb, sem_ib)
    src_wait(0, sa, sem_ia)
    pltpu.async_copy(h_hbm.at[sa.at[0]], rows0, semg0)

    @pl.loop(0, (NCHUNK - 3) // 2)
    def _(p):
        j = 2 * p
        src_wait(j + 1, sb, sem_ib)

        @pl.when(p > 0)
        def _():
            scat_wait(j - 1, rows1, sems1)

        pltpu.async_copy(h_hbm.at[sb.at[0]], rows1, semg1)
        gather_wait(sa, rows0, semg0)
        scat_start(j, rows0, sems0)
        src_load(j + 2, sa, sem_ia)
        src_wait(j + 2, sa, sem_ia)
        scat_wait(j, rows0, sems0)
        pltpu.async_copy(h_hbm.at[sa.at[0]], rows0, semg0)
        gather_wait(sb, rows1, semg1)
        scat_start(j + 1, rows1, sems1)
        src_load(j + 3, sb, sem_ib)

    J = NCHUNK - 3
    src_wait(J + 1, sb, sem_ib)
    scat_wait(J - 1, rows1, sems1)
    pltpu.async_copy(h_hbm.at[sb.at[0]], rows1, semg1)
    gather_wait(sa, rows0, semg0)
    scat_start(J, rows0, sems0)
    src_load(J + 2, sa, sem_ia)
    src_wait(J + 2, sa, sem_ia)
    scat_wait(J, rows0, sems0)
    pltpu.async_copy(h_hbm.at[sa.at[0]], rows0, semg0)
    gather_wait(sb, rows1, semg1)
    scat_start(J + 1, rows1, sems1)
    gather_wait(sa, rows0, semg0)
    scat_wait(J + 1, rows1, sems1)
    pltpu.sync_copy(rows0, agg_sh.at[dst_v.at[J + 2, 0]], add=True)

    plsc.subcore_barrier()

    wbase = s * 624
    pltpu.sync_copy(agg_sh.at[pl.ds(wbase, 624)],
                    out_hbm.at[pl.ds(c * N + wbase, 624)])

    @pl.when(s == NSUB - 1)
    def _():
        pltpu.sync_copy(agg_sh.at[pl.ds(624 * NSUB, N - 624 * NSUB)],
                        out_hbm.at[pl.ds(c * N + 624 * NSUB, N - 624 * NSUB)])


@functools.partial(
    pl.kernel,
    out_type=jax.ShapeDtypeStruct((NSC * N, CH), jnp.float32),
    mesh=plsc.VectorSubcoreMesh(core_axis_name="c", subcore_axis_name="s"),
    scratch_types=[
        pltpu.VMEM_SHARED((AGG_ROWS, CH), jnp.float32),
        pltpu.VMEM((NCHUNK, 1, CHUNK), jnp.int32),
        pltpu.VMEM((1, CHUNK), jnp.int32),
        pltpu.VMEM((1, CHUNK), jnp.int32),
        pltpu.VMEM((CHUNK, CH), jnp.float32),
        pltpu.VMEM((CHUNK, CH), jnp.float32),
        pltpu.SemaphoreType.DMA,
        pltpu.SemaphoreType.DMA,
        pltpu.SemaphoreType.DMA,
        pltpu.SemaphoreType.DMA,
        pltpu.SemaphoreType.DMA,
        pltpu.SemaphoreType.DMA,
    ],
)
def _sc_scatter(ei_hbm, h_hbm, out_hbm, agg_sh, dst_v,
                sa, sb, rows0, rows1,
                sem_ia, sem_ib, semg0, semg1, sems0, sems1):
    _sc_body(ei_hbm, h_hbm, out_hbm, agg_sh, dst_v,
             sa, sb, rows0, rows1,
             sem_ia, sem_ib, semg0, semg1, sems0, sems1)


def _skip_body(x_ref, w2_ref, s_ref):
    s_ref[...] = jnp.dot(x_ref[...], w2_ref[...],
                         preferred_element_type=jnp.float32)


def _skip(x, W2, rb=1000):
    nb = N // rb
    return pl.pallas_call(
        _skip_body,
        grid=(nb,),
        in_specs=[
            pl.BlockSpec((rb, D), lambda i: (i, 0)),
            pl.BlockSpec((D, C), lambda i: (0, 0)),
        ],
        out_specs=pl.BlockSpec((rb, C), lambda i: (i, 0)),
        out_shape=jax.ShapeDtypeStruct((N, C), jnp.float32),
    )(x, W2)


def _out_body(agg_ref, sk_ref, b_ref, ws_ref, bs_ref, o_ref):
    agg = jnp.concatenate([agg_ref[0], agg_ref[1]], axis=-1)
    t = agg + sk_ref[...] + b_ref[...]
    t = jnp.where(t > 0, t, jnp.exp(jnp.minimum(t, 0.0)) - 1.0)
    z = jnp.dot(t, ws_ref[...], preferred_element_type=jnp.float32) + bs_ref[...]
    m = jnp.max(z, axis=-1, keepdims=True)
    e = jnp.exp(z - m)
    o_ref[...] = e / jnp.sum(e, axis=-1, keepdims=True)


def _finish(agg_split, skip, b, Ws, bs, rb=1000):
    nb = N // rb
    return pl.pallas_call(
        _out_body,
        grid=(nb,),
        in_specs=[
            pl.BlockSpec((NSC, rb, CH), lambda i: (0, i, 0)),
            pl.BlockSpec((rb, C), lambda i: (i, 0)),
            pl.BlockSpec((1, C), lambda i: (0, 0)),
            pl.BlockSpec((C, K), lambda i: (0, 0)),
            pl.BlockSpec((1, K), lambda i: (0, 0)),
        ],
        out_specs=pl.BlockSpec((rb, K), lambda i: (i, 0)),
        out_shape=jax.ShapeDtypeStruct((N, K), jnp.float32),
    )(agg_split.reshape(NSC, N, CH), skip, b.reshape(1, C), Ws,
      bs.reshape(1, K))


def kernel(x, edge_index, W1, W2, b, Ws, bs):
    ei5 = edge_index.reshape(2, NSUB, NCHUNK, 1, CHUNK)

    h_split = _h_split(x, W1)
    agg_split = _sc_scatter(ei5, h_split)
    skip = _skip(x, W2)
    return _finish(agg_split, skip, b, Ws, bs)


# speedup vs baseline: 8.6891x; 1.1597x over previous
import functools

import jax
import jax.numpy as jnp
from jax import lax
from jax.experimental import pallas as pl
from jax.experimental.pallas import tpu as pltpu
from jax.experimental.pallas import tpu_sc as plsc

N = 10000
E = 160000
D = 256
C = 256
K = 16

NSC = 2
NSUB = 16
CH = C // NSC
EPS = E // NSUB
CHUNK = 80
NCHUNK = EPS // CHUNK
AGG_ROWS = 10240
RPS = AGG_ROWS // NSUB


def _h_body(x_ref, w1_ref, h_ref):
    hb = jnp.dot(x_ref[...], w1_ref[...], preferred_element_type=jnp.float32)
    h_ref[0] = hb[:, :CH]
    h_ref[1] = hb[:, CH:]


def _h_split(x, W1, rb=1000):
    nb = N // rb
    return pl.pallas_call(
        _h_body,
        grid=(nb,),
        in_specs=[
            pl.BlockSpec((rb, D), lambda i: (i, 0)),
            pl.BlockSpec((D, C), lambda i: (0, 0)),
        ],
        out_specs=pl.BlockSpec((NSC, rb, CH), lambda i: (0, i, 0)),
        out_shape=jax.ShapeDtypeStruct((NSC, N, CH), jnp.float32),
    )(x, W1).reshape(NSC * N, CH)


def _sc_body(ei_hbm, h_hbm, out_hbm, agg_sh, dst_v,
             i0, i1, i2, r0, r1, r2,
             si0, si1, si2, sg0, sg1, sg2, ss0, ss1, ss2):
    c = lax.axis_index("c")
    s = lax.axis_index("s")
    off = c * N
    bufs = [(i0, si0, r0, sg0, ss0),
            (i1, si1, r1, sg1, ss1),
            (i2, si2, r2, sg2, ss2)]

    def src_load(j, iref, sem):
        pltpu.async_copy(ei_hbm.at[0, s, j], iref, sem)

    def src_ready(j, iref, sem):
        pltpu.make_async_copy(ei_hbm.at[0, s, j], iref, sem).wait()
        for k in range(CHUNK // 16):
            iref[0, pl.ds(k * 16, 16)] += off

    def gather_start(iref, rows, sem):
        pltpu.async_copy(h_hbm.at[iref.at[0]], rows, sem)

    def gather_wait(iref, rows, sem):
        pltpu.make_async_copy(h_hbm.at[iref.at[0]], rows, sem).wait()

    def scat_start(j, rows, sem):
        pltpu.async_copy(rows, agg_sh.at[dst_v.at[j, 0]], sem, add=True)

    def scat_wait(j, rows, sem):
        pltpu.make_async_copy(rows, agg_sh.at[dst_v.at[j, 0]], sem).wait()

    pltpu.sync_copy(ei_hbm.at[1, s], dst_v)

    @pl.loop(0, CHUNK)
    def _(i):
        for k in range(CH // 16):
            r0[i, pl.ds(k * 16, 16)] = jnp.zeros((16,), jnp.float32)

    @pl.loop(0, RPS // CHUNK)
    def _(r):
        pltpu.sync_copy(r0, agg_sh.at[pl.ds(s * RPS + r * CHUNK, CHUNK)])

    plsc.subcore_barrier()

    def substep(j, m):
        i_j, si_j, r_j, sg_j, ss_j = bufs[m]
        i_p, si_p, r_p, sg_p, ss_p = bufs[(m + 2) % 3]

        @pl.when(j > 0)
        def _():
            scat_wait(j - 1, r_p, ss_p)

        src_ready(j + 2, i_p, si_p)
        gather_start(i_p, r_p, sg_p)
        gather_wait(i_j, r_j, sg_j)
        scat_start(j, r_j, ss_j)

        @pl.when(j + 3 < NCHUNK)
        def _():
            src_load(j + 3, i_j, si_j)

    src_load(0, i0, si0)
    src_load(1, i1, si1)
    src_load(2, i2, si2)
    src_ready(0, i0, si0)
    gather_start(i0, r0, sg0)
    src_ready(1, i1, si1)
    gather_start(i1, r1, sg1)

    @pl.loop(0, (NCHUNK - 2) // 3)
    def _(q):
        j = 3 * q
        substep(j, 0)
        substep(j + 1, 1)
        substep(j + 2, 2)

    J = NCHUNK - 2
    scat_wait(J - 1, r2, ss2)
    gather_wait(i0, r0, sg0)
    scat_start(J, r0, ss0)
    gather_wait(i1, r1, sg1)
    scat_wait(J, r0, ss0)
    pltpu.sync_copy(r1, agg_sh.at[dst_v.at[J + 1, 0]], add=True)

    plsc.subcore_barrier()

    wbase = s * 624
    pltpu.sync_copy(agg_sh.at[pl.ds(wbase, 624)],
                    out_hbm.at[pl.ds(c * N + wbase, 624)])

    @pl.when(s == NSUB - 1)
    def _():
        pltpu.sync_copy(agg_sh.at[pl.ds(624 * NSUB, N - 624 * NSUB)],
                        out_hbm.at[pl.ds(c * N + 624 * NSUB, N - 624 * NSUB)])


@functools.partial(
    pl.kernel,
    out_type=jax.ShapeDtypeStruct((NSC * N, CH), jnp.float32),
    mesh=plsc.VectorSubcoreMesh(core_axis_name="c", subcore_axis_name="s"),
    scratch_types=(
        [pltpu.VMEM_SHARED((AGG_ROWS, CH), jnp.float32),
         pltpu.VMEM((NCHUNK, 1, CHUNK), jnp.int32)]
        + [pltpu.VMEM((1, CHUNK), jnp.int32)] * 3
        + [pltpu.VMEM((CHUNK, CH), jnp.float32)] * 3
        + [pltpu.SemaphoreType.DMA] * 9
    ),
)
def _sc_scatter(ei_hbm, h_hbm, out_hbm, agg_sh, dst_v,
                i0, i1, i2, r0, r1, r2,
                si0, si1, si2, sg0, sg1, sg2, ss0, ss1, ss2):
    _sc_body(ei_hbm, h_hbm, out_hbm, agg_sh, dst_v,
             i0, i1, i2, r0, r1, r2,
             si0, si1, si2, sg0, sg1, sg2, ss0, ss1, ss2)


def _skip_body(x_ref, w2_ref, s_ref):
    s_ref[...] = jnp.dot(x_ref[...], w2_ref[...],
                         preferred_element_type=jnp.float32)


def _skip(x, W2, rb=1000):
    nb = N // rb
    return pl.pallas_call(
        _skip_body,
        grid=(nb,),
        in_specs=[
            pl.BlockSpec((rb, D), lambda i: (i, 0)),
            pl.BlockSpec((D, C), lambda i: (0, 0)),
        ],
        out_specs=pl.BlockSpec((rb, C), lambda i: (i, 0)),
        out_shape=jax.ShapeDtypeStruct((N, C), jnp.float32),
    )(x, W2)


def _out_body(agg_ref, sk_ref, b_ref, ws_ref, bs_ref, o_ref):
    agg = jnp.concatenate([agg_ref[0], agg_ref[1]], axis=-1)
    t = agg + sk_ref[...] + b_ref[...]
    t = jnp.where(t > 0, t, jnp.exp(jnp.minimum(t, 0.0)) - 1.0)
    z = jnp.dot(t, ws_ref[...], preferred_element_type=jnp.float32) + bs_ref[...]
    m = jnp.max(z, axis=-1, keepdims=True)
    e = jnp.exp(z - m)
    o_ref[...] = e / jnp.sum(e, axis=-1, keepdims=True)


def _finish(agg_split, skip, b, Ws, bs, rb=1000):
    nb = N // rb
    return pl.pallas_call(
        _out_body,
        grid=(nb,),
        in_specs=[
            pl.BlockSpec((NSC, rb, CH), lambda i: (0, i, 0)),
            pl.BlockSpec((rb, C), lambda i: (i, 0)),
            pl.BlockSpec((1, C), lambda i: (0, 0)),
            pl.BlockSpec((C, K), lambda i: (0, 0)),
            pl.BlockSpec((1, K), lambda i: (0, 0)),
        ],
        out_specs=pl.BlockSpec((rb, K), lambda i: (i, 0)),
        out_shape=jax.ShapeDtypeStruct((N, K), jnp.float32),
    )(agg_split.reshape(NSC, N, CH), skip, b.reshape(1, C), Ws,
      bs.reshape(1, K))


def kernel(x, edge_index, W1, W2, b, Ws, bs):
    ei5 = edge_index.reshape(2, NSUB, NCHUNK, 1, CHUNK)

    h_split = _h_split(x, W1)
    agg_split = _sc_scatter(ei5, h_split)
    skip = _skip(x, W2)
    return _finish(agg_split, skip, b, Ws, bs)
